# baseline (speedup 1.0000x reference)
import jax
import jax.numpy as jnp
from jax.experimental import pallas as pl
from jax.experimental.pallas import tpu as pltpu

_B = 1024
_D = 16
_P = 100000
_PBLK = 4096


def _lvq_block(x_ref, pt_ref, out_ref):
    x = x_ref[...]
    pt = pt_ref[...]
    x2 = jnp.sum(x * x, axis=1, keepdims=True)
    p2 = jnp.sum(pt * pt, axis=0, keepdims=True)
    xp = jax.lax.dot_general(
        x, pt, (((1,), (0,)), ((), ())), preferred_element_type=jnp.float32
    )
    sq = x2 + p2 - 2.0 * xp
    out_ref[...] = -jnp.sqrt(jnp.maximum(sq, 1e-12))


@jax.jit
def kernel(x, prototypes):
    pt = prototypes.T
    grid = (pl.cdiv(_P, _PBLK),)
    return pl.pallas_call(
        _lvq_block,
        grid=grid,
        in_specs=[
            pl.BlockSpec((_B, _D), lambda i: (0, 0)),
            pl.BlockSpec((_D, _PBLK), lambda i: (0, i)),
        ],
        out_specs=pl.BlockSpec((_B, _PBLK), lambda i: (0, i)),
        out_shape=jax.ShapeDtypeStruct((_B, _P), jnp.float32),
        compiler_params=pltpu.CompilerParams(
            dimension_semantics=("parallel",),
        ),
    )(x, pt)


# speedup vs baseline: 1.0694x; 1.0694x over previous
import jax
import jax.numpy as jnp
from jax.experimental import pallas as pl
from jax.experimental.pallas import tpu as pltpu

_B = 1024
_D = 16
_P = 100000
_PBLK = 4096


def _lvq_block(xa_ref, pa_ref, out_ref):
    s = jax.lax.dot_general(
        xa_ref[...], pa_ref[...], (((1,), (0,)), ((), ())),
        preferred_element_type=jnp.float32,
    )
    s = jnp.maximum(s, 1e-12)
    out_ref[...] = -(s * jax.lax.rsqrt(s))


@jax.jit
def kernel(x, prototypes):
    x2 = jnp.sum(x * x, axis=1, keepdims=True)
    p2 = jnp.sum(prototypes * prototypes, axis=1, keepdims=True)
    ones_x = jnp.ones((_B, 1), jnp.float32)
    ones_p = jnp.ones((_P, 1), jnp.float32)
    xa = jnp.concatenate([-2.0 * x, x2, ones_x], axis=1)
    pa = jnp.concatenate([prototypes, ones_p, p2], axis=1).T
    grid = (pl.cdiv(_P, _PBLK),)
    return pl.pallas_call(
        _lvq_block,
        grid=grid,
        in_specs=[
            pl.BlockSpec((_B, _D + 2), lambda i: (0, 0)),
            pl.BlockSpec((_D + 2, _PBLK), lambda i: (0, i)),
        ],
        out_specs=pl.BlockSpec((_B, _PBLK), lambda i: (0, i)),
        out_shape=jax.ShapeDtypeStruct((_B, _P), jnp.float32),
        compiler_params=pltpu.CompilerParams(
            dimension_semantics=("parallel",),
        ),
    )(xa, pa)
